# Initial kernel scaffold; baseline (speedup 1.0000x reference)
import jax
import jax.numpy as jnp
from jax.experimental import pallas as pl


def kernel(scores):
    raise NotImplementedError("write your pallas kernel here")


# speedup vs baseline: 296.7455x; 296.7455x over previous
import functools

import jax
import jax.numpy as jnp
from jax.experimental import pallas as pl
from jax.experimental.pallas import tpu as pltpu

B, N, K = 16384, 1001, 10
BLK = 512


def _body(x_ref, num_ref, cnt_ref):
    i = pl.program_id(0)
    x = x_ref[...]
    pos = x[:, 0:1]
    p = jnp.sum((x > pos).astype(jnp.float32), axis=1, keepdims=True)
    col = jax.lax.broadcasted_iota(jnp.int32, (BLK, N), 1)
    num = jnp.zeros((BLK, 1), jnp.float32)
    xc = x
    for r in range(K):
        m = jnp.max(xc, axis=1, keepdims=True)
        fi = jnp.min(jnp.where(xc == m, col, N), axis=1, keepdims=True)
        w = 1.0 / jnp.log2(jnp.float32(r + 2.0))
        bpr = -jnp.log(jnp.clip(jax.nn.sigmoid(pos - m), 1e-8))
        num = num + jnp.where(p == r, 0.0, bpr * w)
        if r < K - 1:
            xc = jnp.where(col == fi, -jnp.inf, xc)
    cnt = 10.0 - (p < K).astype(jnp.float32)

    @pl.when(i == 0)
    def _():
        num_ref[...] = jnp.zeros((1, 1), jnp.float32)
        cnt_ref[...] = jnp.zeros((1, 1), jnp.float32)

    num_ref[...] += jnp.sum(num).reshape(1, 1)
    cnt_ref[...] += jnp.sum(cnt).reshape(1, 1)


def kernel(scores):
    num, cnt = pl.pallas_call(
        _body,
        grid=(B // BLK,),
        in_specs=[pl.BlockSpec((BLK, N), lambda i: (i, 0))],
        out_specs=[
            pl.BlockSpec((1, 1), lambda i: (0, 0)),
            pl.BlockSpec((1, 1), lambda i: (0, 0)),
        ],
        out_shape=[
            jax.ShapeDtypeStruct((1, 1), jnp.float32),
            jax.ShapeDtypeStruct((1, 1), jnp.float32),
        ],
    )(scores)
    return num[0, 0] / jnp.clip(cnt[0, 0], 1.0)
